# baseline (speedup 1.0000x reference)
import functools

import jax
import jax.numpy as jnp
from jax import lax
from jax.experimental import pallas as pl
from jax.experimental.pallas import tpu as pltpu
from jax.experimental.pallas import tpu_sc as plsc

B, S, D = 1, 2048, 768
E, K, F = 8, 2, 3072
TBLK = 256
NTB = S // TBLK
BLK = 128
NA = S * K
NSLOT = NA + E * BLK
NBLK = NSLOT // BLK
NBLK_PAD = 48
FBLK = 768
NF = F // FBLK

NC, NS, L = 2, 16, 16
NW = NC * NS


def _route_body(x_ref, wr_ref, a1_ref, a2_ref, w1_ref, w2_ref, r1_ref,
                r2_ref, cnt_ref, run_ref):
    t = pl.program_id(0)

    @pl.when(t == 0)
    def _init():
        run_ref[...] = jnp.zeros((1, E), jnp.int32)

    xb = x_ref[...]
    logits = jax.lax.dot_general(
        xb, wr_ref[...], (((1,), (1,)), ((), ())),
        preferred_element_type=jnp.float32)
    a1 = jnp.argmax(logits, axis=1)
    m1 = jnp.max(logits, axis=1)
    eids = jax.lax.broadcasted_iota(jnp.int32, (TBLK, E), 1)
    oh1 = eids == a1[:, None]
    masked = jnp.where(oh1, -jnp.inf, logits)
    a2 = jnp.argmax(masked, axis=1)
    m2 = jnp.max(masked, axis=1)
    oh2 = eids == a2[:, None]
    q = jnp.exp(m2 - m1)
    w_top1 = 1.0 / (1.0 + q)
    w_top2 = q / (1.0 + q)

    oh1f = oh1.astype(jnp.float32)
    oh2f = oh2.astype(jnp.float32)
    ohs = oh1f + oh2f
    rows = jax.lax.broadcasted_iota(jnp.int32, (TBLK, TBLK), 0)
    cols = jax.lax.broadcasted_iota(jnp.int32, (TBLK, TBLK), 1)
    tril = (rows > cols).astype(jnp.float32)
    excl = jax.lax.dot_general(tril, ohs, (((1,), (0,)), ((), ())),
                               preferred_element_type=jnp.float32)
    run = run_ref[...]
    r1 = (jnp.sum(excl * oh1f, axis=1).astype(jnp.int32)
          + jnp.sum(jnp.where(oh1, run, 0), axis=1))
    r2 = (jnp.sum(excl * oh2f, axis=1).astype(jnp.int32)
          + jnp.sum(jnp.where(oh2, run, 0), axis=1))

    a1_ref[...] = a1.reshape(1, 1, TBLK)
    a2_ref[...] = a2.reshape(1, 1, TBLK)
    w1_ref[...] = w_top1.reshape(1, 1, TBLK)
    w2_ref[...] = w_top2.reshape(1, 1, TBLK)
    r1_ref[...] = r1.reshape(1, 1, TBLK)
    r2_ref[...] = r2.reshape(1, 1, TBLK)

    run_ref[...] = run + jnp.sum(ohs, axis=0, keepdims=True).astype(jnp.int32)
    cnt_ref[...] = run_ref[...]


def _route(x2, Wr):
    i32 = jnp.int32
    f32 = jnp.float32
    outs = pl.pallas_call(
        _route_body,
        grid=(NTB,),
        in_specs=[
            pl.BlockSpec((TBLK, D), lambda t: (t, 0)),
            pl.BlockSpec((E, D), lambda t: (0, 0)),
        ],
        out_specs=[
            pl.BlockSpec((1, 1, TBLK), lambda t: (t, 0, 0)),
            pl.BlockSpec((1, 1, TBLK), lambda t: (t, 0, 0)),
            pl.BlockSpec((1, 1, TBLK), lambda t: (t, 0, 0)),
            pl.BlockSpec((1, 1, TBLK), lambda t: (t, 0, 0)),
            pl.BlockSpec((1, 1, TBLK), lambda t: (t, 0, 0)),
            pl.BlockSpec((1, 1, TBLK), lambda t: (t, 0, 0)),
            pl.BlockSpec((1, E), lambda t: (0, 0)),
        ],
        out_shape=[
            jax.ShapeDtypeStruct((NTB, 1, TBLK), i32),
            jax.ShapeDtypeStruct((NTB, 1, TBLK), i32),
            jax.ShapeDtypeStruct((NTB, 1, TBLK), f32),
            jax.ShapeDtypeStruct((NTB, 1, TBLK), f32),
            jax.ShapeDtypeStruct((NTB, 1, TBLK), i32),
            jax.ShapeDtypeStruct((NTB, 1, TBLK), i32),
            jax.ShapeDtypeStruct((1, E), i32),
        ],
        scratch_shapes=[pltpu.VMEM((1, E), i32)],
        compiler_params=pltpu.CompilerParams(
            dimension_semantics=("arbitrary",),
        ),
    )(x2, Wr)
    return outs


@functools.cache
def _sc_mesh():
    return plsc.VectorSubcoreMesh(core_axis_name="c", subcore_axis_name="s")


@functools.cache
def _make_dispatch():
    return functools.partial(
        pl.kernel,
        out_type=[
            jax.ShapeDtypeStruct((NSLOT,), jnp.int32),
            jax.ShapeDtypeStruct((NSLOT,), jnp.float32),
            jax.ShapeDtypeStruct((S,), jnp.int32),
            jax.ShapeDtypeStruct((S,), jnp.int32),
            jax.ShapeDtypeStruct((NBLK_PAD,), jnp.int32),
            jax.ShapeDtypeStruct((NBLK_PAD,), jnp.int32),
        ],
        mesh=_sc_mesh(),
        scratch_types=[
            pltpu.VMEM((S,), jnp.int32),
            pltpu.VMEM((S,), jnp.int32),
            pltpu.VMEM((S,), jnp.int32),
            pltpu.VMEM((S,), jnp.int32),
            pltpu.VMEM((S,), jnp.float32),
            pltpu.VMEM((S,), jnp.float32),
            pltpu.VMEM((16,), jnp.int32),
            pltpu.VMEM((16,), jnp.int32),
            pltpu.VMEM((16,), jnp.int32),
            pltpu.VMEM((16,), jnp.int32),
            pltpu.VMEM((NSLOT,), jnp.int32),
            pltpu.VMEM((NSLOT,), jnp.float32),
            pltpu.VMEM((S,), jnp.int32),
            pltpu.VMEM((S,), jnp.int32),
            pltpu.VMEM((NBLK_PAD,), jnp.int32),
            pltpu.VMEM((NBLK_PAD,), jnp.int32),
        ],
        compiler_params=pltpu.CompilerParams(needs_layout_passes=False),
    )(_dispatch_body)


def _dispatch_sc(*args):
    return _make_dispatch()(*args)


def _dispatch_body(a1_hbm, a2_hbm, r1_hbm, r2_hbm, w1_hbm, w2_hbm, cnt_hbm,
                 src_hbm, wsl_hbm, p1_hbm, p2_hbm, bex_hbm, nr_hbm,
                 a1v, a2v, r1v, r2v, w1v, w2v, cntv, padv, offv, csv,
                 srcv, wslv, p1v, p2v, bexv, nrv):
    wid = lax.axis_index("s") * NC + lax.axis_index("c")

    @pl.when(wid == 0)
    def _work():
        pltpu.sync_copy(a1_hbm, a1v)
        pltpu.sync_copy(a2_hbm, a2v)
        pltpu.sync_copy(r1_hbm, r1v)
        pltpu.sync_copy(r2_hbm, r2v)
        pltpu.sync_copy(w1_hbm, w1v)
        pltpu.sync_copy(w2_hbm, w2v)
        pltpu.sync_copy(cnt_hbm, cntv)

        lane = lax.iota(jnp.int32, 16)
        counts = cntv[...]
        padded = (counts + (BLK - 1)) & (-BLK)
        padv[...] = padded
        cs = plsc.cumsum(padded)
        offs = cs - padded
        offv[...] = offs
        csv[...] = cs
        off_sp = [jnp.sum(jnp.where(lane == e, offs, 0)) for e in range(E)]
        cs_sp = [jnp.sum(jnp.where(lane == e, cs, 0)) for e in range(E)]
        end_sp = [jnp.sum(jnp.where(lane == e, offs + counts, 0))
                  for e in range(E)]

        zi = jnp.zeros((16,), jnp.int32)
        zf = jnp.zeros((16,), jnp.float32)

        def zero_body(i, _):
            srcv[pl.ds(i * 16, 16)] = zi
            wslv[pl.ds(i * 16, 16)] = zf
            return _

        lax.fori_loop(0, NSLOT // 16, zero_body, None)

        def _sel(idx16, scalars):
            acc = jnp.zeros((16,), jnp.int32)
            for e in range(E):
                acc = acc + jnp.where(idx16 == e, scalars[e], 0)
            return acc

        def asg_body(c, _):
            base = c * 16
            tok = base + lane
            a1c = a1v[pl.ds(base, 16)]
            p1 = _sel(a1c, off_sp) + r1v[pl.ds(base, 16)]
            plsc.store_scatter(srcv, [p1], tok)
            plsc.store_scatter(wslv, [p1], w1v[pl.ds(base, 16)])
            p1v[pl.ds(base, 16)] = p1
            a2c = a2v[pl.ds(base, 16)]
            p2 = _sel(a2c, off_sp) + r2v[pl.ds(base, 16)]
            plsc.store_scatter(srcv, [p2], tok)
            plsc.store_scatter(wslv, [p2], w2v[pl.ds(base, 16)])
            p2v[pl.ds(base, 16)] = p2
            return _

        lax.fori_loop(0, S // 16, asg_body, None)

        for bi in range(NBLK_PAD // 16):
            sb = (lane + bi * 16) * BLK
            be = jnp.zeros((16,), jnp.int32)
            for e in range(E):
                be = be + jnp.where(cs_sp[e] <= sb, 1, 0)
            be = jnp.minimum(be, E - 1)
            bexv[pl.ds(bi * 16, 16)] = be
            endb = _sel(be, end_sp)
            nrv[pl.ds(bi * 16, 16)] = jnp.clip(endb - sb, 0, BLK)

        pltpu.sync_copy(srcv, src_hbm)
        pltpu.sync_copy(wslv, wsl_hbm)
        pltpu.sync_copy(p1v, p1_hbm)
        pltpu.sync_copy(p2v, p2_hbm)
        pltpu.sync_copy(bexv, bex_hbm)
        pltpu.sync_copy(nrv, nr_hbm)


_G_PER_W = NSLOT // NW
_G_CH = 32
_G_NCH = _G_PER_W // _G_CH
_G_NBUF = 4


@functools.cache
def _make_gather():
    return functools.partial(
        pl.kernel,
        out_type=jax.ShapeDtypeStruct((NSLOT, D), jnp.float32),
        mesh=_sc_mesh(),
        scratch_types=(
            [pltpu.VMEM((_G_PER_W,), jnp.int32)]
            + [pltpu.VMEM((_G_CH, D), jnp.float32)] * _G_NBUF
            + [pltpu.SemaphoreType.DMA] * (2 * _G_NBUF)
        ),
        compiler_params=pltpu.CompilerParams(needs_layout_passes=False),
    )(_gather_body)


def _gather_sc(*args):
    return _make_gather()(*args)


def _gather_body(x_hbm, src_hbm, xbuf_hbm, idxv, *bufs_sems):
    bufs = bufs_sems[:_G_NBUF]
    gsems = bufs_sems[_G_NBUF:2 * _G_NBUF]
    wsems = bufs_sems[2 * _G_NBUF:]
    wid = lax.axis_index("s") * NC + lax.axis_index("c")
    base = wid * _G_PER_W
    pltpu.sync_copy(src_hbm.at[pl.ds(base, _G_PER_W)], idxv)
    gathers = [None] * _G_NCH
    writes = [None] * _G_NCH
    waited = set()
    for c in range(_G_NCH):
        b = c % _G_NBUF
        if c >= _G_NBUF:
            writes[c - _G_NBUF].wait()
            waited.add(c - _G_NBUF)
        gathers[c] = pltpu.async_copy(
            x_hbm.at[idxv.at[pl.ds(c * _G_CH, _G_CH)]], bufs[b], gsems[b])
        d = c - (_G_NBUF - 1)
        if d >= 0:
            gathers[d].wait()
            writes[d] = pltpu.async_copy(
                bufs[d % _G_NBUF],
                xbuf_hbm.at[pl.ds(base + d * _G_CH, _G_CH)],
                wsems[d % _G_NBUF])
    for d in range(_G_NCH):
        if writes[d] is None:
            gathers[d].wait()
            writes[d] = pltpu.async_copy(
                bufs[d % _G_NBUF],
                xbuf_hbm.at[pl.ds(base + d * _G_CH, _G_CH)],
                wsems[d % _G_NBUF])
    for d in range(_G_NCH):
        if d not in waited:
            writes[d].wait()


def _ffn_body(bex_ref, nr_ref, x_ref, src_ref, wsl_ref, w1_ref, w2_ref,
              w3_ref, y_ref, xv, xsem):
    b = pl.program_id(0)
    nreal = nr_ref[b]

    @pl.when(b == 0)
    def _stage_x():
        cp = pltpu.make_async_copy(x_ref, xv, xsem)
        cp.start()
        cp.wait()

    @pl.when(nreal > 0)
    def _compute():
        src_b = src_ref[0, 0, :]
        tids = jax.lax.broadcasted_iota(jnp.int32, (BLK, S), 1)
        sel = (tids == src_b[:, None]).astype(jnp.bfloat16)
        xb = jax.lax.dot_general(sel, xv[...], (((1,), (0,)), ((), ())),
                                 preferred_element_type=jnp.float32)
        w1 = w1_ref[0]
        w3 = w3_ref[0]
        w2 = w2_ref[0]
        h1 = jax.lax.dot_general(xb, w1, (((1,), (1,)), ((), ())),
                                 preferred_element_type=jnp.float32)
        h3 = jax.lax.dot_general(xb, w3, (((1,), (1,)), ((), ())),
                                 preferred_element_type=jnp.float32)
        h = (h1 * jax.nn.sigmoid(h1)) * h3
        y = jax.lax.dot_general(h, w2, (((1,), (1,)), ((), ())),
                                preferred_element_type=jnp.float32)
        y_ref[...] = wsl_ref[0, 0, :][:, None] * y


def _ffn(bex, nr, x2, src3, wsl3, W1, W2, W3):
    grid_spec = pltpu.PrefetchScalarGridSpec(
        num_scalar_prefetch=2,
        grid=(NBLK,),
        in_specs=[
            pl.BlockSpec(memory_space=pl.ANY),
            pl.BlockSpec((1, 1, BLK), lambda b, bex, nr: (b, 0, 0)),
            pl.BlockSpec((1, 1, BLK), lambda b, bex, nr: (b, 0, 0)),
            pl.BlockSpec((1, F, D), lambda b, bex, nr: (bex[b], 0, 0)),
            pl.BlockSpec((1, D, F), lambda b, bex, nr: (bex[b], 0, 0)),
            pl.BlockSpec((1, F, D), lambda b, bex, nr: (bex[b], 0, 0)),
        ],
        out_specs=pl.BlockSpec((BLK, D), lambda b, bex, nr: (b, 0)),
        scratch_shapes=[
            pltpu.VMEM((S, D), jnp.bfloat16),
            pltpu.SemaphoreType.DMA,
        ],
    )
    return pl.pallas_call(
        _ffn_body,
        grid_spec=grid_spec,
        out_shape=jax.ShapeDtypeStruct((NSLOT, D), jnp.float32),
        compiler_params=pltpu.CompilerParams(
            dimension_semantics=("arbitrary",),
            vmem_limit_bytes=110 * 1024 * 1024,
        ),
    )(bex, nr, x2.astype(jnp.bfloat16), src3, wsl3, W1, W2, W3)


_C_PER_W = S // NW
_C_CH = 32
_C_NCH = _C_PER_W // _C_CH


@functools.cache
def _make_combine():
    return functools.partial(
        pl.kernel,
        out_type=jax.ShapeDtypeStruct((S, D), jnp.float32),
        mesh=_sc_mesh(),
        scratch_types=(
            [pltpu.VMEM((_C_PER_W,), jnp.int32)] * 2
            + [pltpu.VMEM((_C_CH, D), jnp.float32)] * (2 * _C_NCH)
            + [pltpu.SemaphoreType.DMA] * (2 * _C_NCH)
            + [pltpu.SemaphoreType.DMA]
        ),
        compiler_params=pltpu.CompilerParams(needs_layout_passes=False),
    )(_combine_body)


def _combine_sc(*args):
    return _make_combine()(*args)


def _combine_body(y_hbm, p1_hbm, p2_hbm, out_hbm, i1v, i2v, *rest):
    r1 = rest[:_C_NCH]
    r2 = rest[_C_NCH:2 * _C_NCH]
    sems = rest[2 * _C_NCH:4 * _C_NCH]
    wsem = rest[4 * _C_NCH]
    wid = lax.axis_index("s") * NC + lax.axis_index("c")
    base = wid * _C_PER_W
    pltpu.sync_copy(p1_hbm.at[pl.ds(base, _C_PER_W)], i1v)
    pltpu.sync_copy(p2_hbm.at[pl.ds(base, _C_PER_W)], i2v)
    cps = []
    for ci in range(_C_NCH):
        sl = pl.ds(ci * _C_CH, _C_CH)
        cps.append((
            pltpu.async_copy(y_hbm.at[i1v.at[sl]], r1[ci], sems[2 * ci]),
            pltpu.async_copy(y_hbm.at[i2v.at[sl]], r2[ci], sems[2 * ci + 1]),
        ))
    writes = []
    for ci in range(_C_NCH):
        cps[ci][0].wait()
        cps[ci][1].wait()

        def add_body(i, _, ci=ci):
            for j in range(D // 16):
                sl = pl.ds(j * 16, 16)
                r1[ci][i, sl] = r1[ci][i, sl] + r2[ci][i, sl]
            return _

        lax.fori_loop(0, _C_CH, add_body, None)
        writes.append(pltpu.async_copy(
            r1[ci], out_hbm.at[pl.ds(base + ci * _C_CH, _C_CH)], wsem))
    for w in writes:
        w.wait()


def kernel(x, Wr, W1, W2, W3):
    x2 = x.reshape(S, D)
    a1, a2, w1, w2, r1, r2, cnt = _route(x2, Wr)
    cnt16 = jnp.concatenate([cnt.reshape(E), jnp.zeros((16 - E,), jnp.int32)])
    src, wsl, p1, p2, bex, nr = _dispatch_sc(
        a1.reshape(S), a2.reshape(S), r1.reshape(S), r2.reshape(S),
        w1.reshape(S), w2.reshape(S), cnt16)
    ybuf = _ffn(bex, nr, x2, src.reshape(NBLK, 1, BLK),
                wsl.reshape(NBLK, 1, BLK), W1, W2, W3)
    out = _combine_sc(ybuf, p1, p2)
    return out.reshape(x.shape)


# speedup vs baseline: 1.4031x; 1.4031x over previous
import functools

import jax
import jax.numpy as jnp
from jax import lax
from jax.experimental import pallas as pl
from jax.experimental.pallas import tpu as pltpu
from jax.experimental.pallas import tpu_sc as plsc

B, S, D = 1, 2048, 768
E, K, F = 8, 2, 3072
TBLK = 512
NTB = S // TBLK
BLK = 256
NA = S * K
NSLOT = NA + E * BLK
NBLK = NSLOT // BLK
NBLK_PAD = 32
FBLK = 768
NF = F // FBLK

NC, NS, L = 2, 16, 16
NW = NC * NS


def _route_body(x_ref, wr_ref, a1_ref, a2_ref, w1_ref, w2_ref, r1_ref,
                r2_ref, cnt_ref, xb16_ref, run_ref):
    t = pl.program_id(0)

    @pl.when(t == 0)
    def _init():
        run_ref[...] = jnp.zeros((1, E), jnp.int32)

    xb = x_ref[...]
    logits = jax.lax.dot_general(
        xb, wr_ref[...], (((1,), (1,)), ((), ())),
        preferred_element_type=jnp.float32)
    a1 = jnp.argmax(logits, axis=1)
    m1 = jnp.max(logits, axis=1)
    eids = jax.lax.broadcasted_iota(jnp.int32, (TBLK, E), 1)
    oh1 = eids == a1[:, None]
    masked = jnp.where(oh1, -jnp.inf, logits)
    a2 = jnp.argmax(masked, axis=1)
    m2 = jnp.max(masked, axis=1)
    oh2 = eids == a2[:, None]
    q = jnp.exp(m2 - m1)
    w_top1 = 1.0 / (1.0 + q)
    w_top2 = q / (1.0 + q)

    oh1f = oh1.astype(jnp.float32)
    oh2f = oh2.astype(jnp.float32)
    ohs = oh1f + oh2f
    rows = jax.lax.broadcasted_iota(jnp.int32, (TBLK, TBLK), 0)
    cols = jax.lax.broadcasted_iota(jnp.int32, (TBLK, TBLK), 1)
    tril = (rows > cols).astype(jnp.float32)
    excl = jax.lax.dot_general(tril, ohs, (((1,), (0,)), ((), ())),
                               preferred_element_type=jnp.float32)
    run = run_ref[...]
    r1 = (jnp.sum(excl * oh1f, axis=1).astype(jnp.int32)
          + jnp.sum(jnp.where(oh1, run, 0), axis=1))
    r2 = (jnp.sum(excl * oh2f, axis=1).astype(jnp.int32)
          + jnp.sum(jnp.where(oh2, run, 0), axis=1))

    a1_ref[...] = a1.reshape(1, 1, TBLK)
    a2_ref[...] = a2.reshape(1, 1, TBLK)
    w1_ref[...] = w_top1.reshape(1, 1, TBLK)
    w2_ref[...] = w_top2.reshape(1, 1, TBLK)
    r1_ref[...] = r1.reshape(1, 1, TBLK)
    r2_ref[...] = r2.reshape(1, 1, TBLK)

    run_ref[...] = run + jnp.sum(ohs, axis=0, keepdims=True).astype(jnp.int32)
    cnt_ref[...] = run_ref[...]
    xb16_ref[...] = xb.astype(jnp.bfloat16)


def _route(x2, Wr):
    i32 = jnp.int32
    f32 = jnp.float32
    outs = pl.pallas_call(
        _route_body,
        grid=(NTB,),
        in_specs=[
            pl.BlockSpec((TBLK, D), lambda t: (t, 0)),
            pl.BlockSpec((E, D), lambda t: (0, 0)),
        ],
        out_specs=[
            pl.BlockSpec((1, 1, TBLK), lambda t: (t, 0, 0)),
            pl.BlockSpec((1, 1, TBLK), lambda t: (t, 0, 0)),
            pl.BlockSpec((1, 1, TBLK), lambda t: (t, 0, 0)),
            pl.BlockSpec((1, 1, TBLK), lambda t: (t, 0, 0)),
            pl.BlockSpec((1, 1, TBLK), lambda t: (t, 0, 0)),
            pl.BlockSpec((1, 1, TBLK), lambda t: (t, 0, 0)),
            pl.BlockSpec((1, E), lambda t: (0, 0)),
            pl.BlockSpec((TBLK, D), lambda t: (t, 0)),
        ],
        out_shape=[
            jax.ShapeDtypeStruct((NTB, 1, TBLK), i32),
            jax.ShapeDtypeStruct((NTB, 1, TBLK), i32),
            jax.ShapeDtypeStruct((NTB, 1, TBLK), f32),
            jax.ShapeDtypeStruct((NTB, 1, TBLK), f32),
            jax.ShapeDtypeStruct((NTB, 1, TBLK), i32),
            jax.ShapeDtypeStruct((NTB, 1, TBLK), i32),
            jax.ShapeDtypeStruct((1, E), i32),
            jax.ShapeDtypeStruct((S, D), jnp.bfloat16),
        ],
        scratch_shapes=[pltpu.VMEM((1, E), i32)],
        compiler_params=pltpu.CompilerParams(
            dimension_semantics=("arbitrary",),
        ),
    )(x2, Wr)
    return outs


@functools.cache
def _sc_mesh():
    return plsc.VectorSubcoreMesh(core_axis_name="c", subcore_axis_name="s")


@functools.cache
def _make_dispatch():
    return functools.partial(
        pl.kernel,
        out_type=[
            jax.ShapeDtypeStruct((NSLOT,), jnp.int32),
            jax.ShapeDtypeStruct((NSLOT,), jnp.float32),
            jax.ShapeDtypeStruct((S,), jnp.int32),
            jax.ShapeDtypeStruct((S,), jnp.int32),
            jax.ShapeDtypeStruct((NBLK_PAD,), jnp.int32),
            jax.ShapeDtypeStruct((NBLK_PAD,), jnp.int32),
        ],
        mesh=_sc_mesh(),
        scratch_types=[
            pltpu.VMEM((S,), jnp.int32),
            pltpu.VMEM((S,), jnp.int32),
            pltpu.VMEM((S,), jnp.int32),
            pltpu.VMEM((S,), jnp.int32),
            pltpu.VMEM((S,), jnp.float32),
            pltpu.VMEM((S,), jnp.float32),
            pltpu.VMEM((16,), jnp.int32),
            pltpu.VMEM((16,), jnp.int32),
            pltpu.VMEM((16,), jnp.int32),
            pltpu.VMEM((16,), jnp.int32),
            pltpu.VMEM((NSLOT,), jnp.int32),
            pltpu.VMEM((NSLOT,), jnp.float32),
            pltpu.VMEM((S,), jnp.int32),
            pltpu.VMEM((S,), jnp.int32),
            pltpu.VMEM((NBLK_PAD,), jnp.int32),
            pltpu.VMEM((NBLK_PAD,), jnp.int32),
        ],
        compiler_params=pltpu.CompilerParams(needs_layout_passes=False),
    )(_dispatch_body)


def _dispatch_sc(*args):
    return _make_dispatch()(*args)


def _dispatch_body(a1_hbm, a2_hbm, r1_hbm, r2_hbm, w1_hbm, w2_hbm, cnt_hbm,
                 src_hbm, wsl_hbm, p1_hbm, p2_hbm, bex_hbm, nr_hbm,
                 a1v, a2v, r1v, r2v, w1v, w2v, cntv, padv, offv, csv,
                 srcv, wslv, p1v, p2v, bexv, nrv):
    wid = lax.axis_index("s") * NC + lax.axis_index("c")

    @pl.when(wid == 0)
    def _work():
        pltpu.sync_copy(a1_hbm, a1v)
        pltpu.sync_copy(a2_hbm, a2v)
        pltpu.sync_copy(r1_hbm, r1v)
        pltpu.sync_copy(r2_hbm, r2v)
        pltpu.sync_copy(w1_hbm, w1v)
        pltpu.sync_copy(w2_hbm, w2v)
        pltpu.sync_copy(cnt_hbm, cntv)

        lane = lax.iota(jnp.int32, 16)
        counts = cntv[...]
        padded = (counts + (BLK - 1)) & (-BLK)
        padv[...] = padded
        cs = plsc.cumsum(padded)
        offs = cs - padded
        offv[...] = offs
        csv[...] = cs
        off_sp = [jnp.sum(jnp.where(lane == e, offs, 0)) for e in range(E)]
        cs_sp = [jnp.sum(jnp.where(lane == e, cs, 0)) for e in range(E)]
        end_sp = [jnp.sum(jnp.where(lane == e, offs + counts, 0))
                  for e in range(E)]

        zi = jnp.zeros((16,), jnp.int32)
        zf = jnp.zeros((16,), jnp.float32)

        def zero_body(i, _):
            srcv[pl.ds(i * 16, 16)] = zi
            wslv[pl.ds(i * 16, 16)] = zf
            return _

        lax.fori_loop(0, NSLOT // 16, zero_body, None)

        def _sel(idx16, scalars):
            acc = jnp.zeros((16,), jnp.int32)
            for e in range(E):
                acc = acc + jnp.where(idx16 == e, scalars[e], 0)
            return acc

        def asg_body(c, _):
            base = c * 16
            tok = base + lane
            a1c = a1v[pl.ds(base, 16)]
            p1 = _sel(a1c, off_sp) + r1v[pl.ds(base, 16)]
            plsc.store_scatter(srcv, [p1], tok)
            plsc.store_scatter(wslv, [p1], w1v[pl.ds(base, 16)])
            p1v[pl.ds(base, 16)] = p1
            a2c = a2v[pl.ds(base, 16)]
            p2 = _sel(a2c, off_sp) + r2v[pl.ds(base, 16)]
            plsc.store_scatter(srcv, [p2], tok)
            plsc.store_scatter(wslv, [p2], w2v[pl.ds(base, 16)])
            p2v[pl.ds(base, 16)] = p2
            return _

        lax.fori_loop(0, S // 16, asg_body, None)

        for bi in range(NBLK_PAD // 16):
            sb = (lane + bi * 16) * BLK
            be = jnp.zeros((16,), jnp.int32)
            for e in range(E):
                be = be + jnp.where(cs_sp[e] <= sb, 1, 0)
            be = jnp.minimum(be, E - 1)
            bexv[pl.ds(bi * 16, 16)] = be
            endb = _sel(be, end_sp)
            nrv[pl.ds(bi * 16, 16)] = jnp.clip(endb - sb, 0, BLK)

        pltpu.sync_copy(srcv, src_hbm)
        pltpu.sync_copy(wslv, wsl_hbm)
        pltpu.sync_copy(p1v, p1_hbm)
        pltpu.sync_copy(p2v, p2_hbm)
        pltpu.sync_copy(bexv, bex_hbm)
        pltpu.sync_copy(nrv, nr_hbm)


_G_PER_W = NSLOT // NW
_G_CH = 32
_G_NCH = _G_PER_W // _G_CH
_G_NBUF = 4


@functools.cache
def _make_gather():
    return functools.partial(
        pl.kernel,
        out_type=jax.ShapeDtypeStruct((NSLOT, D), jnp.float32),
        mesh=_sc_mesh(),
        scratch_types=(
            [pltpu.VMEM((_G_PER_W,), jnp.int32)]
            + [pltpu.VMEM((_G_CH, D), jnp.float32)] * _G_NBUF
            + [pltpu.SemaphoreType.DMA] * (2 * _G_NBUF)
        ),
        compiler_params=pltpu.CompilerParams(needs_layout_passes=False),
    )(_gather_body)


def _gather_sc(*args):
    return _make_gather()(*args)


def _gather_body(x_hbm, src_hbm, xbuf_hbm, idxv, *bufs_sems):
    bufs = bufs_sems[:_G_NBUF]
    gsems = bufs_sems[_G_NBUF:2 * _G_NBUF]
    wsems = bufs_sems[2 * _G_NBUF:]
    wid = lax.axis_index("s") * NC + lax.axis_index("c")
    base = wid * _G_PER_W
    pltpu.sync_copy(src_hbm.at[pl.ds(base, _G_PER_W)], idxv)
    gathers = [None] * _G_NCH
    writes = [None] * _G_NCH
    waited = set()
    for c in range(_G_NCH):
        b = c % _G_NBUF
        if c >= _G_NBUF:
            writes[c - _G_NBUF].wait()
            waited.add(c - _G_NBUF)
        gathers[c] = pltpu.async_copy(
            x_hbm.at[idxv.at[pl.ds(c * _G_CH, _G_CH)]], bufs[b], gsems[b])
        d = c - (_G_NBUF - 1)
        if d >= 0:
            gathers[d].wait()
            writes[d] = pltpu.async_copy(
                bufs[d % _G_NBUF],
                xbuf_hbm.at[pl.ds(base + d * _G_CH, _G_CH)],
                wsems[d % _G_NBUF])
    for d in range(_G_NCH):
        if writes[d] is None:
            gathers[d].wait()
            writes[d] = pltpu.async_copy(
                bufs[d % _G_NBUF],
                xbuf_hbm.at[pl.ds(base + d * _G_CH, _G_CH)],
                wsems[d % _G_NBUF])
    for d in range(_G_NCH):
        if d not in waited:
            writes[d].wait()


def _ffn_body(bex_ref, nr_ref, x_ref, src_ref, wsl_ref, w1_ref, w2_ref,
              w3_ref, y_ref, xv, xsem):
    b = pl.program_id(0)
    nreal = nr_ref[b]

    @pl.when(b == 0)
    def _stage_x():
        cp = pltpu.make_async_copy(x_ref, xv, xsem)
        cp.start()
        cp.wait()

    @pl.when(nreal > 0)
    def _compute():
        src_b = src_ref[0, 0, :]
        tids = jax.lax.broadcasted_iota(jnp.int32, (BLK, S), 1)
        sel = (tids == src_b[:, None]).astype(jnp.bfloat16)
        xb = jax.lax.dot_general(sel, xv[...], (((1,), (0,)), ((), ())),
                                 preferred_element_type=jnp.float32)
        w1 = w1_ref[0]
        w3 = w3_ref[0]
        w2 = w2_ref[0]
        h1 = jax.lax.dot_general(xb, w1, (((1,), (1,)), ((), ())),
                                 preferred_element_type=jnp.float32)
        h3 = jax.lax.dot_general(xb, w3, (((1,), (1,)), ((), ())),
                                 preferred_element_type=jnp.float32)
        h = (h1 * jax.nn.sigmoid(h1)) * h3
        y = jax.lax.dot_general(h, w2, (((1,), (1,)), ((), ())),
                                preferred_element_type=jnp.float32)
        y_ref[...] = wsl_ref[0, 0, :][:, None] * y


def _ffn(bex, nr, x2, src3, wsl3, W1, W2, W3):
    grid_spec = pltpu.PrefetchScalarGridSpec(
        num_scalar_prefetch=2,
        grid=(NBLK,),
        in_specs=[
            pl.BlockSpec(memory_space=pl.ANY),
            pl.BlockSpec((1, 1, BLK), lambda b, bex, nr: (b, 0, 0)),
            pl.BlockSpec((1, 1, BLK), lambda b, bex, nr: (b, 0, 0)),
            pl.BlockSpec((1, F, D), lambda b, bex, nr: (bex[b], 0, 0)),
            pl.BlockSpec((1, D, F), lambda b, bex, nr: (bex[b], 0, 0)),
            pl.BlockSpec((1, F, D), lambda b, bex, nr: (bex[b], 0, 0)),
        ],
        out_specs=pl.BlockSpec((BLK, D), lambda b, bex, nr: (b, 0)),
        scratch_shapes=[
            pltpu.VMEM((S, D), jnp.bfloat16),
            pltpu.SemaphoreType.DMA,
        ],
    )
    return pl.pallas_call(
        _ffn_body,
        grid_spec=grid_spec,
        out_shape=jax.ShapeDtypeStruct((NSLOT, D), jnp.float32),
        compiler_params=pltpu.CompilerParams(
            dimension_semantics=("arbitrary",),
            vmem_limit_bytes=110 * 1024 * 1024,
        ),
    )(bex, nr, x2, src3, wsl3, W1, W2, W3)


_C_PER_W = S // NW
_C_CH = 32
_C_NCH = _C_PER_W // _C_CH


@functools.cache
def _make_combine():
    return functools.partial(
        pl.kernel,
        out_type=jax.ShapeDtypeStruct((S, D), jnp.float32),
        mesh=_sc_mesh(),
        scratch_types=(
            [pltpu.VMEM((_C_PER_W,), jnp.int32)] * 2
            + [pltpu.VMEM((_C_CH, D), jnp.float32)] * (2 * _C_NCH)
            + [pltpu.SemaphoreType.DMA] * (2 * _C_NCH)
            + [pltpu.SemaphoreType.DMA]
        ),
        compiler_params=pltpu.CompilerParams(needs_layout_passes=False),
    )(_combine_body)


def _combine_sc(*args):
    return _make_combine()(*args)


def _combine_body(y_hbm, p1_hbm, p2_hbm, out_hbm, i1v, i2v, *rest):
    r1 = rest[:_C_NCH]
    r2 = rest[_C_NCH:2 * _C_NCH]
    sems = rest[2 * _C_NCH:4 * _C_NCH]
    wsem = rest[4 * _C_NCH]
    wid = lax.axis_index("s") * NC + lax.axis_index("c")
    base = wid * _C_PER_W
    pltpu.sync_copy(p1_hbm.at[pl.ds(base, _C_PER_W)], i1v)
    pltpu.sync_copy(p2_hbm.at[pl.ds(base, _C_PER_W)], i2v)
    cps = []
    for ci in range(_C_NCH):
        sl = pl.ds(ci * _C_CH, _C_CH)
        cps.append((
            pltpu.async_copy(y_hbm.at[i1v.at[sl]], r1[ci], sems[2 * ci]),
            pltpu.async_copy(y_hbm.at[i2v.at[sl]], r2[ci], sems[2 * ci + 1]),
        ))
    writes = []
    for ci in range(_C_NCH):
        cps[ci][0].wait()
        cps[ci][1].wait()

        def add_body(i, _, ci=ci):
            for j in range(D // 16):
                sl = pl.ds(j * 16, 16)
                r1[ci][i, sl] = r1[ci][i, sl] + r2[ci][i, sl]
            return _

        lax.fori_loop(0, _C_CH, add_body, None)
        writes.append(pltpu.async_copy(
            r1[ci], out_hbm.at[pl.ds(base + ci * _C_CH, _C_CH)], wsem))
    for w in writes:
        w.wait()


def kernel(x, Wr, W1, W2, W3):
    x2 = x.reshape(S, D)
    a1, a2, w1, w2, r1, r2, cnt, x16 = _route(x2, Wr)
    cnt16 = jnp.concatenate([cnt.reshape(E), jnp.zeros((16 - E,), jnp.int32)])
    src, wsl, p1, p2, bex, nr = _dispatch_sc(
        a1.reshape(S), a2.reshape(S), r1.reshape(S), r2.reshape(S),
        w1.reshape(S), w2.reshape(S), cnt16)
    ybuf = _ffn(bex, nr, x16, src.reshape(NBLK, 1, BLK),
                wsl.reshape(NBLK, 1, BLK), W1, W2, W3)
    out = _combine_sc(ybuf, p1, p2)
    return out.reshape(x.shape)
